# Initial kernel scaffold; baseline (speedup 1.0000x reference)
import jax
import jax.numpy as jnp
from jax.experimental import pallas as pl


def kernel(x_nchw, w_oihw, gamma, beta):
    raise NotImplementedError("write your pallas kernel here")


# speedup vs baseline: 2.0430x; 2.0430x over previous
import functools

import jax
import jax.numpy as jnp
from jax import lax
from jax.experimental import pallas as pl
from jax.experimental.pallas import tpu as pltpu

_N, _C, _H, _W = 32, 64, 56, 56
_HP = _H + 3
_WP = 64
_TH = 8
_NB = _H // _TH
_MB = _TH * _WP
_VMEM_LIMIT = 48 * 1024 * 1024


def _conv_stats_kernel(x_ref, w_ref, y_ref, st_ref, xcat_ref):
    nx = _H * _WP + _TH
    xf = x_ref[0].reshape(_HP * _WP, _C)
    xcat_ref[...] = jnp.concatenate(
        [xf[0:nx], xf[_WP:_WP + nx], xf[2 * _WP:2 * _WP + nx]], axis=1)
    col = lax.broadcasted_iota(jnp.int32, (_MB, _C), 0) % _WP
    valid = col < _W
    ssum = jnp.zeros((1, _C), jnp.float32)
    ssq = jnp.zeros((1, _C), jnp.float32)
    for b in range(_NB):
        acc = jnp.zeros((_MB, _C), jnp.float32)
        for dx in range(3):
            a = xcat_ref[pl.ds(b * _MB + dx, _MB), :]
            acc = acc + jnp.dot(a, w_ref[dx],
                                preferred_element_type=jnp.float32)
        y_ref[0, pl.ds(b * _TH, _TH)] = acc.reshape(_TH, _WP, _C)
        m = jnp.where(valid, acc, 0.0)
        ssum = ssum + jnp.sum(m, axis=0, keepdims=True)
        ssq = ssq + jnp.sum(m * m, axis=0, keepdims=True)
    st_ref[...] = jnp.concatenate([ssum, ssq], axis=0).reshape(1, 2, _C)


def _bn_relu_kernel(y_ref, sc_ref, sh_ref, o_ref):
    o_ref[...] = jnp.maximum(y_ref[...] * sc_ref[0, :] + sh_ref[0, :], 0.0)


def kernel(x_nchw, w_oihw, gamma, beta, eps=1e-5):
    n, c, h, w = x_nchw.shape
    x = jnp.transpose(x_nchw, (0, 2, 3, 1))
    xp = jnp.pad(x, ((0, 0), (1, _HP - h - 1), (1, _WP - w - 1), (0, 0)))
    xp = xp.astype(jnp.bfloat16)
    wcat = (jnp.transpose(w_oihw, (3, 2, 1, 0))
            .reshape(3, 3 * c, c).astype(jnp.bfloat16))

    flops = 2 * n * h * _WP * 9 * c * c
    y, st = pl.pallas_call(
        _conv_stats_kernel,
        out_shape=(jax.ShapeDtypeStruct((n, h, _WP, c), jnp.float32),
                   jax.ShapeDtypeStruct((n, 2, c), jnp.float32)),
        grid=(n,),
        in_specs=[
            pl.BlockSpec((1, _HP, _WP, c), lambda i: (i, 0, 0, 0)),
            pl.BlockSpec((3, 3 * c, c), lambda i: (0, 0, 0)),
        ],
        out_specs=(
            pl.BlockSpec((1, h, _WP, c), lambda i: (i, 0, 0, 0)),
            pl.BlockSpec((1, 2, c), lambda i: (i, 0, 0)),
        ),
        scratch_shapes=[pltpu.VMEM((_H * _WP + _TH, 3 * c), jnp.bfloat16)],
        compiler_params=pltpu.CompilerParams(
            dimension_semantics=("parallel",),
            vmem_limit_bytes=_VMEM_LIMIT),
        cost_estimate=pl.CostEstimate(
            flops=flops, transcendentals=0,
            bytes_accessed=2 * n * _HP * _WP * c + 4 * n * h * _WP * c),
    )(xp, wcat)

    count = float(n * h * w)
    tot = jnp.sum(st, axis=0)
    mean = tot[0] / count
    var = jnp.maximum(tot[1] / count - mean * mean, 0.0)
    inv = lax.rsqrt(var + eps)
    scale = (gamma.astype(jnp.float32) * inv).reshape(1, c)
    shift = (beta.astype(jnp.float32) - mean * gamma * inv).reshape(1, c)

    out = pl.pallas_call(
        _bn_relu_kernel,
        out_shape=jax.ShapeDtypeStruct(y.shape, jnp.float32),
        grid=(n,),
        in_specs=[
            pl.BlockSpec((1, h, _WP, c), lambda i: (i, 0, 0, 0)),
            pl.BlockSpec((1, c), lambda i: (0, 0)),
            pl.BlockSpec((1, c), lambda i: (0, 0)),
        ],
        out_specs=pl.BlockSpec((1, h, _WP, c), lambda i: (i, 0, 0, 0)),
        compiler_params=pltpu.CompilerParams(
            dimension_semantics=("parallel",),
            vmem_limit_bytes=_VMEM_LIMIT),
    )(y, scale, shift)

    return jnp.transpose(out[:, :, :w, :], (0, 3, 1, 2))


# speedup vs baseline: 2.0469x; 1.0019x over previous
import functools

import jax
import jax.numpy as jnp
from jax import lax
from jax.experimental import pallas as pl
from jax.experimental.pallas import tpu as pltpu

_N, _C, _H, _W = 32, 64, 56, 56
_HW = _H * _W
_WP = 64
_TH = 8
_NB = _H // _TH
_MB = _TH * _WP
_MT = _H * _WP
_NPAD = (_H + 3) * _WP
_VMEM_LIMIT = 48 * 1024 * 1024


def _conv_stats_kernel(x_ref, w_ref, y_ref, st_ref, xcat_ref):
    xt = x_ref[0].astype(jnp.bfloat16).T
    xpf = jnp.pad(xt.reshape(_H, _W, _C),
                  ((1, 2), (1, _WP - _W - 1), (0, 0))).reshape(_NPAD, _C)
    nx = _MT + _TH
    xcat_ref[...] = jnp.concatenate(
        [xpf[0:nx], xpf[_WP:_WP + nx], xpf[2 * _WP:2 * _WP + nx]], axis=1)
    col = lax.broadcasted_iota(jnp.int32, (_MB, _C), 0) % _WP
    valid = col < _W
    ssum = jnp.zeros((1, _C), jnp.float32)
    ssq = jnp.zeros((1, _C), jnp.float32)
    for b in range(_NB):
        acc = jnp.zeros((_MB, _C), jnp.float32)
        for dx in range(3):
            a = xcat_ref[pl.ds(b * _MB + dx, _MB), :]
            acc = acc + jnp.dot(a, w_ref[dx],
                                preferred_element_type=jnp.float32)
        y_ref[0, pl.ds(b * _MB, _MB)] = acc.astype(jnp.bfloat16)
        m = jnp.where(valid, acc, 0.0)
        ssum = ssum + jnp.sum(m, axis=0, keepdims=True)
        ssq = ssq + jnp.sum(m * m, axis=0, keepdims=True)
    st_ref[...] = jnp.concatenate([ssum, ssq], axis=0).reshape(1, 2, _C)


def _bn_relu_t_kernel(y_ref, st_ref, g_ref, b_ref, o_ref, *, eps):
    count = float(_N * _H * _W)
    tot = jnp.sum(st_ref[...], axis=0)
    mean = tot[0] / count
    var = jnp.maximum(tot[1] / count - mean * mean, 0.0)
    inv = lax.rsqrt(var + eps)
    sc = (g_ref[0, :] * inv).reshape(1, _C)
    sh = (b_ref[0, :] - mean * g_ref[0, :] * inv).reshape(1, _C)
    o = jnp.maximum(y_ref[0].astype(jnp.float32) * sc + sh, 0.0)
    o = o.reshape(_H, _WP, _C)[:, :_W, :].reshape(_HW, _C)
    o_ref[0] = o.T


def kernel(x_nchw, w_oihw, gamma, beta, eps=1e-5):
    n, c, h, w = x_nchw.shape
    xf = x_nchw.reshape(n, c, h * w)
    wcat = (jnp.transpose(w_oihw, (3, 2, 1, 0))
            .reshape(3, 3 * c, c).astype(jnp.bfloat16))

    flops = 2 * n * h * _WP * 9 * c * c
    y, st = pl.pallas_call(
        _conv_stats_kernel,
        out_shape=(jax.ShapeDtypeStruct((n, _MT, c), jnp.bfloat16),
                   jax.ShapeDtypeStruct((n, 2, c), jnp.float32)),
        grid=(n,),
        in_specs=[
            pl.BlockSpec((1, c, h * w), lambda i: (i, 0, 0)),
            pl.BlockSpec((3, 3 * c, c), lambda i: (0, 0, 0)),
        ],
        out_specs=(
            pl.BlockSpec((1, _MT, c), lambda i: (i, 0, 0)),
            pl.BlockSpec((1, 2, c), lambda i: (i, 0, 0)),
        ),
        scratch_shapes=[pltpu.VMEM((_MT + _TH, 3 * c), jnp.bfloat16)],
        compiler_params=pltpu.CompilerParams(
            dimension_semantics=("parallel",),
            vmem_limit_bytes=_VMEM_LIMIT),
        cost_estimate=pl.CostEstimate(
            flops=flops, transcendentals=0,
            bytes_accessed=4 * n * c * h * w + 2 * n * _MT * c),
    )(xf, wcat)

    out = pl.pallas_call(
        functools.partial(_bn_relu_t_kernel, eps=eps),
        out_shape=jax.ShapeDtypeStruct((n, c, h * w), jnp.float32),
        grid=(n,),
        in_specs=[
            pl.BlockSpec((1, _MT, c), lambda i: (i, 0, 0)),
            pl.BlockSpec((n, 2, c), lambda i: (0, 0, 0)),
            pl.BlockSpec((1, c), lambda i: (0, 0)),
            pl.BlockSpec((1, c), lambda i: (0, 0)),
        ],
        out_specs=pl.BlockSpec((1, c, h * w), lambda i: (i, 0, 0)),
        compiler_params=pltpu.CompilerParams(
            dimension_semantics=("parallel",),
            vmem_limit_bytes=_VMEM_LIMIT),
    )(y, st, gamma.reshape(1, c), beta.reshape(1, c))

    return out.reshape(n, c, h, w)
